# Initial kernel scaffold; baseline (speedup 1.0000x reference)
import jax
import jax.numpy as jnp
from jax.experimental import pallas as pl


def kernel(inputs, W, b, W_attn, A_tilde, edges):
    raise NotImplementedError("write your pallas kernel here")


# speedup vs baseline: 26.0791x; 26.0791x over previous
import functools

import jax
import jax.numpy as jnp
from jax import lax
from jax.experimental import pallas as pl
from jax.experimental.pallas import tpu as pltpu
from jax.experimental.pallas import tpu_sc as plsc

BN = 512
EBLK = 512
EGRP = 128
NSUB = 16
NCORE = 2


def _tc_embed(x_pad, W, wa_mat, n_pad, F, EMB):
    def body(x_ref, w_ref, wa_ref, nst_ref, s_ref):
        nst = jnp.dot(x_ref[...], w_ref[...], preferred_element_type=jnp.float32)
        nst_ref[...] = nst
        s_ref[...] = jnp.dot(nst, wa_ref[...], preferred_element_type=jnp.float32)

    return pl.pallas_call(
        body,
        grid=(n_pad // BN,),
        in_specs=[
            pl.BlockSpec((BN, F), lambda i: (i, 0)),
            pl.BlockSpec((F, EMB), lambda i: (0, 0)),
            pl.BlockSpec((EMB, 2), lambda i: (0, 0)),
        ],
        out_specs=[
            pl.BlockSpec((BN, EMB), lambda i: (i, 0)),
            pl.BlockSpec((BN, 2), lambda i: (i, 0)),
        ],
        out_shape=[
            jax.ShapeDtypeStruct((n_pad, EMB), jnp.float32),
            jax.ShapeDtypeStruct((n_pad, 2), jnp.float32),
        ],
    )(x_pad, W, wa_mat)


def _tc_combine(p0, p1, b2d, n_pad, EMB):
    def body(a_ref, c_ref, b_ref, o_ref):
        o_ref[...] = a_ref[...] + c_ref[...] + b_ref[...]

    return pl.pallas_call(
        body,
        grid=(n_pad // BN,),
        in_specs=[
            pl.BlockSpec((BN, EMB), lambda i: (i, 0)),
            pl.BlockSpec((BN, EMB), lambda i: (i, 0)),
            pl.BlockSpec((1, EMB), lambda i: (0, 0)),
        ],
        out_specs=pl.BlockSpec((BN, EMB), lambda i: (i, 0)),
        out_shape=jax.ShapeDtypeStruct((n_pad, EMB), jnp.float32),
    )(p0, p1, b2d)


def _make_sc_kernel(n_pad, e_pad, EMB):
    chunk1 = e_pad // NSUB
    chunk2 = e_pad // (NSUB * NCORE)
    nb1 = chunk1 // EBLK
    nb2 = chunk2 // EBLK
    ngrp = EBLK // EGRP
    rpt = n_pad // NSUB
    mesh = plsc.VectorSubcoreMesh(core_axis_name="c", subcore_axis_name="s")

    def body(rows_hbm, cols_hbm, ssrc_hbm, sdst_hbm, nst_hbm, zseg_hbm, zout_hbm,
             out_hbm,
             ssrc_v, sdst_v, seg_v, ridx_v, cidx_v, score_v, rows_v,
             spmem_seg, spmem_out, sem):
        ci = lax.axis_index("c")
        si = lax.axis_index("s")
        wid = si * NCORE + ci

        @pl.when(si == 0)
        def _zero_seg():
            pltpu.sync_copy(zseg_hbm, spmem_seg)

        pltpu.sync_copy(zout_hbm.at[pl.ds(si * rpt, rpt)],
                        spmem_out.at[pl.ds(si * rpt, rpt)])
        pltpu.sync_copy(ssrc_hbm, ssrc_v)
        pltpu.sync_copy(sdst_hbm, sdst_v)
        plsc.subcore_barrier()

        def edge_scores(jg):
            j, o = jg // (EGRP // 16), (jg % (EGRP // 16)) * 16
            r_reg = ridx_v[j, pl.ds(o, 16)]
            c_reg = cidx_v[j, pl.ds(o, 16)]
            t = plsc.load_gather(ssrc_v, [r_reg]) + plsc.load_gather(sdst_v, [c_reg])
            t = jnp.where(t >= 0.0, t, 0.2 * t)
            t = jnp.clip(t, -2.0, 2.0)
            return j, o, r_reg, jnp.exp(t)

        def p1_block(blk, carry):
            base128 = si * (chunk1 // EGRP) + blk * ngrp
            pltpu.sync_copy(rows_hbm.at[pl.ds(base128, ngrp)], ridx_v)
            pltpu.sync_copy(cols_hbm.at[pl.ds(base128, ngrp)], cidx_v)
            for jg in range(EBLK // 16):
                j, o, _, sc = edge_scores(jg)
                score_v[j, pl.ds(o, 16)] = sc
            for j in range(ngrp):
                pltpu.sync_copy(score_v.at[j], spmem_seg.at[ridx_v.at[j]], add=True)
            return carry

        lax.fori_loop(0, nb1, p1_block, 0)
        plsc.subcore_barrier()
        pltpu.sync_copy(spmem_seg, seg_v)

        def p2_block(blk, carry):
            base128 = wid * (chunk2 // EGRP) + blk * ngrp
            pltpu.sync_copy(rows_hbm.at[pl.ds(base128, ngrp)], ridx_v)
            pltpu.sync_copy(cols_hbm.at[pl.ds(base128, ngrp)], cidx_v)
            copies = [
                pltpu.async_copy(nst_hbm.at[cidx_v.at[j]],
                                 rows_v.at[pl.ds(j * EGRP, EGRP)], sem)
                for j in range(ngrp)
            ]
            for jg in range(EBLK // 16):
                j, o, r_reg, sc = edge_scores(jg)
                score_v[j, pl.ds(o, 16)] = sc / plsc.load_gather(seg_v, [r_reg])
            for d in copies:
                d.wait()

            def scale_grp(g, c2):
                for j in range(ngrp):
                    a_reg = score_v[j, pl.ds(g * 16, 16)]
                    for l in range(16):
                        a = jnp.full((16,), a_reg[l], jnp.float32)
                        row = j * EGRP + g * 16 + l
                        for f in range(EMB // 16):
                            rows_v[row, pl.ds(f * 16, 16)] = (
                                rows_v[row, pl.ds(f * 16, 16)] * a)
                return c2

            lax.fori_loop(0, EGRP // 16, scale_grp, 0)
            for j in range(ngrp):
                pltpu.sync_copy(rows_v.at[pl.ds(j * EGRP, EGRP)],
                                spmem_out.at[ridx_v.at[j]], add=True)
            return carry

        lax.fori_loop(0, nb2, p2_block, 0)
        plsc.subcore_barrier()
        pltpu.sync_copy(spmem_out.at[pl.ds(si * rpt, rpt)],
                        out_hbm.at[ci, pl.ds(si * rpt, rpt)])

    return pl.kernel(
        body,
        out_type=jax.ShapeDtypeStruct((NCORE, n_pad, EMB), jnp.float32),
        mesh=mesh,
        compiler_params=pltpu.CompilerParams(
            needs_layout_passes=False, use_tc_tiling_on_sc=False),
        scratch_types=[
            pltpu.VMEM((n_pad,), jnp.float32),
            pltpu.VMEM((n_pad,), jnp.float32),
            pltpu.VMEM((n_pad,), jnp.float32),
            pltpu.VMEM((EBLK // EGRP, EGRP), jnp.int32),
            pltpu.VMEM((EBLK // EGRP, EGRP), jnp.int32),
            pltpu.VMEM((EBLK // EGRP, EGRP), jnp.float32),
            pltpu.VMEM((EBLK, EMB), jnp.float32),
            pltpu.VMEM_SHARED((n_pad,), jnp.float32),
            pltpu.VMEM_SHARED((n_pad, EMB), jnp.float32),
            pltpu.SemaphoreType.DMA,
        ],
    )


def kernel(inputs, W, b, W_attn, A_tilde, edges):
    B, N, F = inputs.shape
    EMB = W.shape[1]
    E = edges.shape[0]
    del A_tilde

    n_pad = ((N + BN - 1) // BN) * BN
    ebt = EBLK * NSUB * NCORE
    e_pad = ((E + ebt - 1) // ebt) * ebt

    x_pad = jnp.pad(inputs[0], ((0, n_pad - N), (0, 0)))
    wa_mat = W_attn.reshape(2, EMB).T

    nst, s = _tc_embed(x_pad, W, wa_mat, n_pad, F, EMB)
    ssrc = s[:, 0]
    sdst = s[:, 1]

    pad_idx = jnp.full((e_pad - E,), N, jnp.int32)
    rows2d = jnp.concatenate([edges[:, 0].astype(jnp.int32), pad_idx]).reshape(-1, EGRP)
    cols2d = jnp.concatenate([edges[:, 1].astype(jnp.int32), pad_idx]).reshape(-1, EGRP)
    zseg = jnp.zeros((n_pad,), jnp.float32)
    zout = jnp.zeros((n_pad, EMB), jnp.float32)

    parts = _make_sc_kernel(n_pad, e_pad, EMB)(
        rows2d, cols2d, ssrc, sdst, nst, zseg, zout)

    out = _tc_combine(parts[0], parts[1], b.reshape(1, EMB), n_pad, EMB)
    return out[:N].reshape(B, N, EMB)
